# Initial kernel scaffold; baseline (speedup 1.0000x reference)
import jax
import jax.numpy as jnp
from jax.experimental import pallas as pl


def kernel(query_source, context, W_q, W_kv, W_out, g_ctx, b_ctx, g_qs, b_qs, g_out, b_out, gamma):
    raise NotImplementedError("write your pallas kernel here")


# speedup vs baseline: 2.7707x; 2.7707x over previous
import jax
import jax.numpy as jnp
from jax.experimental import pallas as pl
from jax.experimental.pallas import tpu as pltpu

HEADS = 8
DIM_HEAD = 64
DIM = 384
INNER = HEADS * DIM_HEAD
HW = 32
N = HW * HW
TOP = 5
NSEL = TOP * TOP
SELPAD = 32

_DIMS = (((1,), (0,)), ((), ()))
_DIMS_T = (((0,), (0,)), ((), ()))


def _mm(a, b):
    return jax.lax.dot_general(a, b, _DIMS, preferred_element_type=jnp.float32)


def _mm_t(a, b):
    return jax.lax.dot_general(a, b, _DIMS_T,
                               preferred_element_type=jnp.float32)


def _mm_hi(a, b):
    return jax.lax.dot_general(a, b, _DIMS,
                               precision=jax.lax.Precision.HIGHEST,
                               preferred_element_type=jnp.float32)


def _chan_ln(x, g, b, eps=1e-5):
    mean = jnp.mean(x, axis=0, keepdims=True)
    var = jnp.mean(jnp.abs(x - mean) ** 2, axis=0, keepdims=True)
    return (x - mean) / jnp.sqrt(var + eps) * g + b


def _l2norm(t):
    n = jnp.sqrt(jnp.sum(t * t, axis=0, keepdims=True))
    return t / jnp.maximum(n, 1e-12)


def _top5_hots(score, lane):
    idxs = []
    s = score
    for _ in range(TOP):
        m = jnp.max(s, axis=1, keepdims=True)
        idx = jnp.min(jnp.where(s >= m, lane, HW), axis=1, keepdims=True)
        idxs.append(idx)
        s = jnp.where(lane == idx, -3e38, s)
    return idxs


def _head_attend(qh, kh, vh, consts):
    R2, C2, lane32, jcol, sub32, i32 = consts
    qn = _l2norm(qh)
    kn = _l2norm(kh)

    probe = jnp.sum(qn, axis=1, keepdims=True)
    kabs = jnp.abs(kn)
    hsum = _mm_hi(kabs, R2)
    wsum = _mm_hi(kabs, C2)
    score_r = _mm_t(probe, hsum)
    score_c = _mm_t(probe, wsum)
    ridx = _top5_hots(score_r, lane32)
    cidx = _top5_hots(score_c, lane32)

    A = jnp.zeros((N, SELPAD), jnp.float32)
    Bm = jnp.zeros((N, SELPAD), jnp.float32)
    for a in range(TOP):
        amask = ((i32 // TOP) == a) & (i32 < NSEL)
        A = A + jnp.where(amask & ((jcol // HW) == ridx[a]), 1.0, 0.0)
        bmask = ((i32 % TOP) == a) & (i32 < NSEL)
        Bm = Bm + jnp.where(bmask & ((jcol % HW) == cidx[a]), 1.0, 0.0)
    P = A * Bm

    k_sel = _mm(kn, P)
    v_sel = _mm(vh, P)

    sim = _mm_t(k_sel, qn)
    sim = jnp.where(sub32 < NSEL, sim, -3e38)
    m = jnp.max(sim, axis=0, keepdims=True)
    e = jnp.exp(sim - m)
    attn = e / jnp.sum(e, axis=0, keepdims=True)
    return _mm(v_sel, attn)


def _sel_consts():
    j32 = jax.lax.broadcasted_iota(jnp.int32, (N, SELPAD), 0)
    i32 = jax.lax.broadcasted_iota(jnp.int32, (N, SELPAD), 1)
    R2 = (((j32 // HW) == i32) & (i32 < HW)).astype(jnp.float32)
    C2 = (((j32 % HW) == i32) & (i32 < HW)).astype(jnp.float32)
    lane32 = jax.lax.broadcasted_iota(jnp.int32, (1, HW), 1)
    jcol = jax.lax.broadcasted_iota(jnp.int32, (N, 1), 0)
    sub32 = jax.lax.broadcasted_iota(jnp.int32, (SELPAD, 1), 0)
    return R2, C2, lane32, jcol, sub32, i32


def _fused(qs_ref, cx_ref, wq_ref, wkv_ref, wo_ref,
           gc_ref, bc_ref, gq_ref, bq_ref, go_ref, bo_ref, gamma_ref,
           o_ref):
    qs_n = _chan_ln(qs_ref[0], gq_ref[...], bq_ref[...])
    cx_n = _chan_ln(cx_ref[0], gc_ref[...], bc_ref[...])

    q = _mm(wq_ref[...], qs_n)
    kv = _mm(wkv_ref[...], cx_n)

    consts = _sel_consts()
    outs = []
    for h in range(HEADS):
        s = slice(h * DIM_HEAD, (h + 1) * DIM_HEAD)
        outs.append(_head_attend(
            q[s, :], kv[s, :],
            kv[INNER + h * DIM_HEAD:INNER + (h + 1) * DIM_HEAD, :], consts))

    inner = jnp.concatenate(outs, axis=0)
    out = _mm(wo_ref[...], inner)
    out = _chan_ln(out, go_ref[...], bo_ref[...])
    o_ref[0] = gamma_ref[0] * out + qs_n


@jax.jit
def kernel(query_source, context, W_q, W_kv, W_out,
           g_ctx, b_ctx, g_qs, b_qs, g_out, b_out, gamma):
    B, C, H, W = query_source.shape
    qs = query_source.reshape(B, C, H * W)
    cx = context.reshape(B, C, H * W)

    col = lambda t: t.reshape(C, 1)
    full = lambda shape: pl.BlockSpec(shape, lambda b: (0,) * len(shape))

    out = pl.pallas_call(
        _fused,
        grid=(B,),
        in_specs=[
            pl.BlockSpec((1, C, H * W), lambda b: (b, 0, 0)),
            pl.BlockSpec((1, C, H * W), lambda b: (b, 0, 0)),
            full((INNER, C)),
            full((2 * INNER, C)),
            full((C, INNER)),
            full((C, 1)), full((C, 1)), full((C, 1)),
            full((C, 1)), full((C, 1)), full((C, 1)),
            pl.BlockSpec(memory_space=pltpu.SMEM),
        ],
        out_specs=pl.BlockSpec((1, C, H * W), lambda b: (b, 0, 0)),
        out_shape=jax.ShapeDtypeStruct((B, C, H * W), jnp.float32),
    )(qs, cx, W_q, W_kv, W_out,
      col(g_ctx), col(b_ctx), col(g_qs), col(b_qs), col(g_out), col(b_out),
      gamma)
    return out.reshape(B, C, H, W)


# speedup vs baseline: 4.6277x; 1.6702x over previous
import jax
import jax.numpy as jnp
from jax.experimental import pallas as pl
from jax.experimental.pallas import tpu as pltpu

HEADS = 8
DIM_HEAD = 64
DIM = 384
INNER = HEADS * DIM_HEAD
HW = 32
N = HW * HW
TOP = 5
NSEL = TOP * TOP
SELPAD = 32

_DIMS = (((1,), (0,)), ((), ()))
_DIMS_T = (((0,), (0,)), ((), ()))


def _mm(a, b):
    return jax.lax.dot_general(a, b, _DIMS, preferred_element_type=jnp.float32)


def _mm_t(a, b):
    return jax.lax.dot_general(a, b, _DIMS_T,
                               preferred_element_type=jnp.float32)


def _mm_hi(a, b):
    return jax.lax.dot_general(a, b, _DIMS,
                               precision=jax.lax.Precision.HIGHEST,
                               preferred_element_type=jnp.float32)


def _chan_ln(x, g, b, eps=1e-5):
    mean = jnp.mean(x, axis=0, keepdims=True)
    var = jnp.mean(jnp.abs(x - mean) ** 2, axis=0, keepdims=True)
    return (x - mean) / jnp.sqrt(var + eps) * g + b


def _l2norm(t):
    n = jnp.sqrt(jnp.sum(t * t, axis=0, keepdims=True))
    return t / jnp.maximum(n, 1e-12)


def _top5_batched(scores, lane):
    idxs = []
    s = scores
    for _ in range(TOP):
        m = jnp.max(s, axis=1, keepdims=True)
        idx = jnp.min(jnp.where(s >= m, lane, HW), axis=1, keepdims=True)
        idxs.append(idx)
        s = jnp.where(lane == idx, -3e38, s)
    return idxs


def _fused(qs_ref, cx_ref, wq_ref, wkv_ref, wo_ref,
           gc_ref, bc_ref, gq_ref, bq_ref, go_ref, bo_ref, gamma_ref,
           o_ref):
    qs_n = _chan_ln(qs_ref[0], gq_ref[...], bq_ref[...])
    cx_n = _chan_ln(cx_ref[0], gc_ref[...], bc_ref[...])

    q = _mm(wq_ref[...], qs_n)
    kv = _mm(wkv_ref[...], cx_n)

    j64 = jax.lax.broadcasted_iota(jnp.int32, (N, 2 * HW), 0)
    i64 = jax.lax.broadcasted_iota(jnp.int32, (N, 2 * HW), 1)
    RC2 = (((j64 // HW) == i64) | ((j64 % HW) == (i64 - HW))).astype(
        jnp.float32)
    lane32 = jax.lax.broadcasted_iota(jnp.int32, (1, HW), 1)
    jrow = jax.lax.broadcasted_iota(jnp.int32, (N, 1), 0) // HW
    jmod = jax.lax.broadcasted_iota(jnp.int32, (N, 1), 0) % HW
    sub32 = jax.lax.broadcasted_iota(jnp.int32, (SELPAD, 1), 0)

    qn = [_l2norm(q[h * DIM_HEAD:(h + 1) * DIM_HEAD, :]) for h in range(HEADS)]
    kn = [_l2norm(kv[h * DIM_HEAD:(h + 1) * DIM_HEAD, :]) for h in range(HEADS)]
    probe = [jnp.sum(t, axis=1, keepdims=True) for t in qn]

    kabs_all = jnp.concatenate([jnp.abs(t) for t in kn], axis=0)
    RS = _mm_hi(kabs_all, RC2)

    scores = jnp.concatenate(
        [_mm_t(probe[h], RS[h * DIM_HEAD:(h + 1) * DIM_HEAD, :HW])
         for h in range(HEADS)] +
        [_mm_t(probe[h], RS[h * DIM_HEAD:(h + 1) * DIM_HEAD, HW:])
         for h in range(HEADS)], axis=0)

    idxs = _top5_batched(scores, lane32)
    TRr = jnp.zeros((2 * HEADS, SELPAD), jnp.int32)
    TRc = jnp.zeros((2 * HEADS, SELPAD), jnp.int32)
    for a in range(TOP):
        TRr = TRr + jnp.where((lane32 // TOP) == a, idxs[a], 0)
        TRc = TRc + jnp.where((lane32 % TOP) == a, idxs[a], 0)
    TRr = jnp.where(lane32 < NSEL, TRr, -1)
    TRc = jnp.where(lane32 < NSEL, TRc, -1)

    outs = []
    for h in range(HEADS):
        vh = kv[INNER + h * DIM_HEAD:INNER + (h + 1) * DIM_HEAD, :]
        P = jnp.where((jrow == TRr[h:h + 1, :]) &
                      (jmod == TRc[HEADS + h:HEADS + h + 1, :]),
                      1.0, 0.0)

        k_sel = _mm(kn[h], P)
        v_sel = _mm(vh, P)

        sim = _mm_t(k_sel, qn[h])
        sim = jnp.where(sub32 < NSEL, sim, -3e38)
        m = jnp.max(sim, axis=0, keepdims=True)
        e = jnp.exp(sim - m)
        attn = e / jnp.sum(e, axis=0, keepdims=True)
        outs.append(_mm(v_sel, attn))

    inner = jnp.concatenate(outs, axis=0)
    out = _mm(wo_ref[...], inner)
    out = _chan_ln(out, go_ref[...], bo_ref[...])
    o_ref[0] = gamma_ref[0] * out + qs_n


@jax.jit
def kernel(query_source, context, W_q, W_kv, W_out,
           g_ctx, b_ctx, g_qs, b_qs, g_out, b_out, gamma):
    B, C, H, W = query_source.shape
    qs = query_source.reshape(B, C, H * W)
    cx = context.reshape(B, C, H * W)

    col = lambda t: t.reshape(C, 1)
    full = lambda shape: pl.BlockSpec(shape, lambda b: (0,) * len(shape))

    out = pl.pallas_call(
        _fused,
        grid=(B,),
        in_specs=[
            pl.BlockSpec((1, C, H * W), lambda b: (b, 0, 0)),
            pl.BlockSpec((1, C, H * W), lambda b: (b, 0, 0)),
            full((INNER, C)),
            full((2 * INNER, C)),
            full((C, INNER)),
            full((C, 1)), full((C, 1)), full((C, 1)),
            full((C, 1)), full((C, 1)), full((C, 1)),
            pl.BlockSpec(memory_space=pltpu.SMEM),
        ],
        out_specs=pl.BlockSpec((1, C, H * W), lambda b: (b, 0, 0)),
        out_shape=jax.ShapeDtypeStruct((B, C, H * W), jnp.float32),
    )(qs, cx, W_q, W_kv, W_out,
      col(g_ctx), col(b_ctx), col(g_qs), col(b_qs), col(g_out), col(b_out),
      gamma)
    return out.reshape(B, C, H, W)
